# baseline (speedup 1.0000x reference)
import jax
import jax.numpy as jnp
from jax.experimental import pallas as pl


def _leaky(x):
    return jnp.where(x > 0, x, 0.2 * x)


def _rowdot(a, b):
    return jax.lax.dot_general(a, b, (((1,), (1,)), ((), ())),
                               preferred_element_type=jnp.float32)


def _mm(a, b):
    return jnp.dot(a, b, preferred_element_type=jnp.float32)


def _split3(x):
    h1 = x.astype(jnp.bfloat16).astype(jnp.float32)
    r = x - h1
    h2 = r.astype(jnp.bfloat16).astype(jnp.float32)
    h3 = r - h2
    return h1, h2, h3


def _gather(onehot, table):
    c = table.shape[1] // 3
    g = jnp.dot(onehot, table, preferred_element_type=jnp.float32)
    return (g[:, :c] + g[:, c:2 * c]) + g[:, 2 * c:]


def _tree3(p0, p1, p2):
    return (p0 + p2) + p1


def _norm3(x):
    return _tree3(x[:, 0:1] * x[:, 0:1], x[:, 1:2] * x[:, 1:2],
                  x[:, 2:3] * x[:, 2:3])


def _lfa_tile(qxyz, rxyz, ftrans, Wn, bn, Wan, ba, k, cdim):
    tq = qxyz.shape[0]
    n = rxyz.shape[0]
    q2 = _norm3(qxyz)
    r2 = _norm3(rxyz)
    d = (q2 - 2.0 * _rowdot(qxyz, rxyz)) + r2.reshape(1, n)
    fp = _split3(ftrans)
    xp = _split3(rxyz)
    tab = jnp.concatenate(list(fp) + list(xp), axis=1).astype(jnp.bfloat16)
    xb = 3 * cdim
    col = jax.lax.broadcasted_iota(jnp.int32, (tq, n), 1)
    ohs = []
    for _ in range(k):
        m = jnp.min(d, axis=1, keepdims=True)
        cand = jnp.where(d == m, col, n)
        sel = jnp.min(cand, axis=1, keepdims=True)
        hit = col == sel
        ohs.append(hit.astype(jnp.bfloat16))
        d = jnp.where(hit, jnp.inf, d)
    oh = jnp.concatenate(ohs, axis=0)
    g = jnp.dot(oh, tab, preferred_element_type=jnp.float32)
    h = (g[:, :cdim] + g[:, cdim:2 * cdim]) + g[:, 2 * cdim:xb]
    nbr = (g[:, xb:xb + 3] + g[:, xb + 3:xb + 6]) + g[:, xb + 6:xb + 9]
    qrep = jnp.concatenate([qxyz] * k, axis=0)
    rel = nbr - qrep
    dist = jnp.sqrt(_tree3(rel[:, 0:1] * rel[:, 0:1],
                           rel[:, 1:2] * rel[:, 1:2],
                           rel[:, 2:3] * rel[:, 2:3]) + 1e-12)
    geo = jnp.concatenate([rel, dist, qrep, nbr], axis=1)
    nf = _leaky(_mm(geo, Wn) + bn)
    z = (h + _mm(nf, Wan)) + ba
    acc = z[:tq]
    for j in range(1, k):
        acc = jnp.maximum(acc, z[j * tq:(j + 1) * tq])
    return _leaky(acc)


def kernel(xyz, feature, enc_xyz, enc_feature, Wn0, bn0, Wa0, ba0, Wc0, bc0,
           Wf0, bf0, Wn1, bn1, Wa1, ba1, Wc1, bc1):
    B, N, _ = xyz.shape
    N2 = 2 * N
    TQ0 = N // 8
    TQB = N2 // 8
    TQC = N2 // 16

    Wa0f, Wa0n = Wa0[:256], Wa0[256:]
    Wa1f, Wa1n = Wa1[:128], Wa1[128:]
    bn0r, ba0r = bn0.reshape(1, -1), ba0.reshape(1, -1)
    bn1r, ba1r = bn1.reshape(1, -1), ba1.reshape(1, -1)
    bc0r, bf0r, bc1r = bc0.reshape(1, -1), bf0.reshape(1, -1), bc1.reshape(1, -1)

    full = lambda shape: pl.BlockSpec(shape, lambda b, t, s=len(shape): (0,) * s)

    def stage_a(xyzt_ref, xyzf_ref, feat_ref, Wn_ref, bn_ref, Waf_ref,
                Wan_ref, ba_ref, Wc_ref, bc_ref, f0_ref, c6_ref):
        qxyz = xyzt_ref[0]
        rxyz = xyzf_ref[0]
        ftrans = _mm(feat_ref[0], Waf_ref[...])
        f0 = _lfa_tile(qxyz, rxyz, ftrans, Wn_ref[...], bn_ref[...],
                       Wan_ref[...], ba_ref[...], 8, 256)
        f0_ref[0] = f0
        offa = _mm(f0[:, :128], Wc_ref[...]) + bc_ref[...]
        offb = _mm(f0[:, 128:], Wc_ref[...]) + bc_ref[...]
        c6_ref[0] = jnp.concatenate([qxyz + offa, qxyz + offb], axis=1)

    na = N // TQ0
    f0, c6 = pl.pallas_call(
        stage_a,
        grid=(B, na),
        in_specs=[
            pl.BlockSpec((1, TQ0, 3), lambda b, t: (b, t, 0)),
            pl.BlockSpec((1, N, 3), lambda b, t: (b, 0, 0)),
            pl.BlockSpec((1, N, 256), lambda b, t: (b, 0, 0)),
            full((10, 32)), full((1, 32)), full((256, 256)), full((32, 256)),
            full((1, 256)), full((128, 3)), full((1, 3)),
        ],
        out_specs=[
            pl.BlockSpec((1, TQ0, 256), lambda b, t: (b, t, 0)),
            pl.BlockSpec((1, TQ0, 6), lambda b, t: (b, t, 0)),
        ],
        out_shape=[
            jax.ShapeDtypeStruct((B, N, 256), jnp.float32),
            jax.ShapeDtypeStruct((B, N, 6), jnp.float32),
        ],
    )(xyz, xyz, feature, Wn0, bn0r, Wa0f, Wa0n, ba0r, Wc0, bc0r)

    xyz1 = c6.reshape(B, N2, 3)
    f_up = f0.reshape(B, N2, 128)

    def stage_b(x1_ref, f_ref, ex_ref, ef_ref, Wf_ref, bf_ref, Waf_ref,
                qt_ref, loss_ref):
        qxyz = x1_ref[0]
        e = ex_ref[0]
        tq = qxyz.shape[0]
        ne = e.shape[0]
        q2 = _norm3(qxyz)
        e2 = _norm3(e)
        d = (q2 - 2.0 * _rowdot(qxyz, e)) + e2.reshape(1, ne)
        etab = jnp.concatenate(list(_split3(ef_ref[0])),
                               axis=1).astype(jnp.bfloat16)
        col = jax.lax.broadcasted_iota(jnp.int32, (tq, ne), 1)
        ohs = []
        for _ in range(4):
            m = jnp.min(d, axis=1, keepdims=True)
            cand = jnp.where(d == m, col, ne)
            sel = jnp.min(cand, axis=1, keepdims=True)
            hit = col == sel
            ohs.append(hit.astype(jnp.bfloat16))
            d = jnp.where(hit, jnp.inf, d)
        oh = jnp.concatenate(ohs, axis=0)
        gr = jnp.dot(oh, etab, preferred_element_type=jnp.float32)
        rr = (gr[:, :128] + gr[:, 128:256]) + gr[:, 256:384]
        res = jnp.maximum(jnp.maximum(rr[:tq], rr[tq:2 * tq]),
                          jnp.maximum(rr[2 * tq:3 * tq], rr[3 * tq:]))
        pred = _mm(f_ref[0], Wf_ref[...]) + bf_ref[...]
        loc = pred[:, :128]
        log_b = pred[:, 128:]
        bb = jnp.exp(jnp.clip(log_b, -8.0, 8.0)) + 1e-6
        qv = jnp.round(res)

        def cdf(x):
            return 0.5 + 0.5 * jnp.sign(x - loc) * (
                1.0 - jnp.exp(-jnp.abs(x - loc) / bb))

        p = jnp.clip(cdf(qv + 0.5) - cdf(qv - 0.5), 1e-9, 1.0)
        tile_loss = jnp.sum(-jnp.log2(p), keepdims=True)

        @pl.when(jnp.logical_and(pl.program_id(0) == 0, pl.program_id(1) == 0))
        def _():
            loss_ref[...] = jnp.zeros((1, 1), jnp.float32)

        loss_ref[...] += tile_loss
        qt_ref[0] = _mm(qv, Waf_ref[...])

    nb = N2 // TQB
    nc = N2 // TQC
    qt, loss_sum = pl.pallas_call(
        stage_b,
        grid=(B, nb),
        in_specs=[
            pl.BlockSpec((1, TQB, 3), lambda b, t: (b, t, 0)),
            pl.BlockSpec((1, TQB, 128), lambda b, t: (b, t, 0)),
            pl.BlockSpec((1, N2, 3), lambda b, t: (b, 0, 0)),
            pl.BlockSpec((1, N2, 128), lambda b, t: (b, 0, 0)),
            full((128, 256)), full((1, 256)), full((128, 128)),
        ],
        out_specs=[
            pl.BlockSpec((1, TQB, 128), lambda b, t: (b, t, 0)),
            pl.BlockSpec((1, 1), lambda b, t: (0, 0)),
        ],
        out_shape=[
            jax.ShapeDtypeStruct((B, N2, 128), jnp.float32),
            jax.ShapeDtypeStruct((1, 1), jnp.float32),
        ],
    )(xyz1, f_up, enc_xyz, enc_feature, Wf0, bf0r, Wa1f)

    loss = (loss_sum[0, 0] / (B * N2 * 128)).reshape(())

    def stage_c(x1t_ref, x1f_ref, qt_ref, Wn_ref, bn_ref, Wan_ref, ba_ref,
                Wc_ref, bc_ref, c6_ref, fo_ref):
        qxyz = x1t_ref[0]
        rxyz = x1f_ref[0]
        f1 = _lfa_tile(qxyz, rxyz, qt_ref[0], Wn_ref[...], bn_ref[...],
                       Wan_ref[...], ba_ref[...], 8, 128)
        fo_ref[0] = f1
        offa = _mm(f1[:, :64], Wc_ref[...]) + bc_ref[...]
        offb = _mm(f1[:, 64:], Wc_ref[...]) + bc_ref[...]
        c6_ref[0] = jnp.concatenate([qxyz + offa, qxyz + offb], axis=1)

    c6b, f1 = pl.pallas_call(
        stage_c,
        grid=(B, nc),
        in_specs=[
            pl.BlockSpec((1, TQC, 3), lambda b, t: (b, t, 0)),
            pl.BlockSpec((1, N2, 3), lambda b, t: (b, 0, 0)),
            pl.BlockSpec((1, N2, 128), lambda b, t: (b, 0, 0)),
            full((10, 32)), full((1, 32)), full((32, 128)), full((1, 128)),
            full((64, 3)), full((1, 3)),
        ],
        out_specs=[
            pl.BlockSpec((1, TQC, 6), lambda b, t: (b, t, 0)),
            pl.BlockSpec((1, TQC, 128), lambda b, t: (b, t, 0)),
        ],
        out_shape=[
            jax.ShapeDtypeStruct((B, N2, 6), jnp.float32),
            jax.ShapeDtypeStruct((B, N2, 128), jnp.float32),
        ],
    )(xyz1, xyz1, qt, Wn1, bn1r, Wa1n, ba1r, Wc1, bc1r)

    coord2 = c6b.reshape(B, 4 * N, 3)
    fout = f1.reshape(B, 4 * N, 64)
    return (coord2, fout, loss)


# speedup vs baseline: 1.0550x; 1.0550x over previous
import jax
import jax.numpy as jnp
from jax.experimental import pallas as pl


def _leaky(x):
    return jnp.where(x > 0, x, 0.2 * x)


def _rowdot(a, b):
    return jax.lax.dot_general(a, b, (((1,), (1,)), ((), ())),
                               preferred_element_type=jnp.float32)


def _mm(a, b):
    return jnp.dot(a, b, preferred_element_type=jnp.float32)


def _split3(x):
    h1 = x.astype(jnp.bfloat16).astype(jnp.float32)
    r = x - h1
    h2 = r.astype(jnp.bfloat16).astype(jnp.float32)
    h3 = r - h2
    return h1, h2, h3


def _gather(onehot, table):
    c = table.shape[1] // 3
    g = jnp.dot(onehot, table, preferred_element_type=jnp.float32)
    return (g[:, :c] + g[:, c:2 * c]) + g[:, 2 * c:]


def _tree3(p0, p1, p2):
    return (p0 + p2) + p1


def _norm3(x):
    return _tree3(x[:, 0:1] * x[:, 0:1], x[:, 1:2] * x[:, 1:2],
                  x[:, 2:3] * x[:, 2:3])


def _lfa_tile(qxyz, rxyz, ftrans, Wn, bn, Wan, ba, k, cdim):
    tq = qxyz.shape[0]
    n = rxyz.shape[0]
    q2 = _norm3(qxyz)
    r2 = _norm3(rxyz)
    d = (q2 - 2.0 * _rowdot(qxyz, rxyz)) + r2.reshape(1, n)
    fp = _split3(ftrans)
    xp = _split3(rxyz)
    tab = jnp.concatenate(list(fp) + list(xp), axis=1).astype(jnp.bfloat16)
    xb = 3 * cdim
    col = jax.lax.broadcasted_iota(jnp.int32, (tq, n), 1)
    acc = jnp.full((tq, cdim), -jnp.inf, jnp.float32)
    for _ in range(k):
        m = jnp.min(d, axis=1, keepdims=True)
        cand = jnp.where(d == m, col, n)
        sel = jnp.min(cand, axis=1, keepdims=True)
        hit = col == sel
        onehot = hit.astype(jnp.bfloat16)
        d = jnp.where(hit, jnp.inf, d)
        g = jnp.dot(onehot, tab, preferred_element_type=jnp.float32)
        h = (g[:, :cdim] + g[:, cdim:2 * cdim]) + g[:, 2 * cdim:xb]
        nbr = (g[:, xb:xb + 3] + g[:, xb + 3:xb + 6]) + g[:, xb + 6:xb + 9]
        rel = nbr - qxyz
        dist = jnp.sqrt(_tree3(rel[:, 0:1] * rel[:, 0:1],
                               rel[:, 1:2] * rel[:, 1:2],
                               rel[:, 2:3] * rel[:, 2:3]) + 1e-12)
        geo = jnp.concatenate([rel, dist, qxyz, nbr], axis=1)
        nf = _leaky(_mm(geo, Wn) + bn)
        z = (h + _mm(nf, Wan)) + ba
        acc = jnp.maximum(acc, z)
    return _leaky(acc)


def kernel(xyz, feature, enc_xyz, enc_feature, Wn0, bn0, Wa0, ba0, Wc0, bc0,
           Wf0, bf0, Wn1, bn1, Wa1, ba1, Wc1, bc1):
    B, N, _ = xyz.shape
    N2 = 2 * N
    TQ0 = N // 4
    TQB = N2 // 8
    TQC = N2 // 8

    Wa0f, Wa0n = Wa0[:256], Wa0[256:]
    Wa1f, Wa1n = Wa1[:128], Wa1[128:]
    bn0r, ba0r = bn0.reshape(1, -1), ba0.reshape(1, -1)
    bn1r, ba1r = bn1.reshape(1, -1), ba1.reshape(1, -1)
    bc0r, bf0r, bc1r = bc0.reshape(1, -1), bf0.reshape(1, -1), bc1.reshape(1, -1)

    full = lambda shape: pl.BlockSpec(shape, lambda b, t, s=len(shape): (0,) * s)

    def stage_a(xyzt_ref, xyzf_ref, feat_ref, Wn_ref, bn_ref, Waf_ref,
                Wan_ref, ba_ref, Wc_ref, bc_ref, f0_ref, c6_ref):
        qxyz = xyzt_ref[0]
        rxyz = xyzf_ref[0]
        ftrans = _mm(feat_ref[0], Waf_ref[...])
        f0 = _lfa_tile(qxyz, rxyz, ftrans, Wn_ref[...], bn_ref[...],
                       Wan_ref[...], ba_ref[...], 8, 256)
        f0_ref[0] = f0
        offa = _mm(f0[:, :128], Wc_ref[...]) + bc_ref[...]
        offb = _mm(f0[:, 128:], Wc_ref[...]) + bc_ref[...]
        c6_ref[0] = jnp.concatenate([qxyz + offa, qxyz + offb], axis=1)

    na = N // TQ0
    f0, c6 = pl.pallas_call(
        stage_a,
        grid=(B, na),
        in_specs=[
            pl.BlockSpec((1, TQ0, 3), lambda b, t: (b, t, 0)),
            pl.BlockSpec((1, N, 3), lambda b, t: (b, 0, 0)),
            pl.BlockSpec((1, N, 256), lambda b, t: (b, 0, 0)),
            full((10, 32)), full((1, 32)), full((256, 256)), full((32, 256)),
            full((1, 256)), full((128, 3)), full((1, 3)),
        ],
        out_specs=[
            pl.BlockSpec((1, TQ0, 256), lambda b, t: (b, t, 0)),
            pl.BlockSpec((1, TQ0, 6), lambda b, t: (b, t, 0)),
        ],
        out_shape=[
            jax.ShapeDtypeStruct((B, N, 256), jnp.float32),
            jax.ShapeDtypeStruct((B, N, 6), jnp.float32),
        ],
    )(xyz, xyz, feature, Wn0, bn0r, Wa0f, Wa0n, ba0r, Wc0, bc0r)

    xyz1 = c6.reshape(B, N2, 3)
    f_up = f0.reshape(B, N2, 128)

    def stage_b(x1_ref, f_ref, ex_ref, ef_ref, Wf_ref, bf_ref, Waf_ref,
                qt_ref, loss_ref):
        qxyz = x1_ref[0]
        e = ex_ref[0]
        tq = qxyz.shape[0]
        ne = e.shape[0]
        q2 = _norm3(qxyz)
        e2 = _norm3(e)
        d = (q2 - 2.0 * _rowdot(qxyz, e)) + e2.reshape(1, ne)
        etab = jnp.concatenate(list(_split3(ef_ref[0])),
                               axis=1).astype(jnp.bfloat16)
        col = jax.lax.broadcasted_iota(jnp.int32, (tq, ne), 1)
        res = jnp.full((tq, 128), -jnp.inf, jnp.float32)
        for _ in range(4):
            m = jnp.min(d, axis=1, keepdims=True)
            cand = jnp.where(d == m, col, ne)
            sel = jnp.min(cand, axis=1, keepdims=True)
            hit = col == sel
            d = jnp.where(hit, jnp.inf, d)
            gr = jnp.dot(hit.astype(jnp.bfloat16), etab,
                         preferred_element_type=jnp.float32)
            res = jnp.maximum(res, (gr[:, :128] + gr[:, 128:256])
                              + gr[:, 256:384])
        pred = _mm(f_ref[0], Wf_ref[...]) + bf_ref[...]
        loc = pred[:, :128]
        log_b = pred[:, 128:]
        bb = jnp.exp(jnp.clip(log_b, -8.0, 8.0)) + 1e-6
        qv = jnp.round(res)

        def cdf(x):
            return 0.5 + 0.5 * jnp.sign(x - loc) * (
                1.0 - jnp.exp(-jnp.abs(x - loc) / bb))

        p = jnp.clip(cdf(qv + 0.5) - cdf(qv - 0.5), 1e-9, 1.0)
        tile_loss = jnp.sum(-jnp.log2(p), keepdims=True)

        @pl.when(jnp.logical_and(pl.program_id(0) == 0, pl.program_id(1) == 0))
        def _():
            loss_ref[...] = jnp.zeros((1, 1), jnp.float32)

        loss_ref[...] += tile_loss
        qt_ref[0] = _mm(qv, Waf_ref[...])

    nb = N2 // TQB
    nc = N2 // TQC
    qt, loss_sum = pl.pallas_call(
        stage_b,
        grid=(B, nb),
        in_specs=[
            pl.BlockSpec((1, TQB, 3), lambda b, t: (b, t, 0)),
            pl.BlockSpec((1, TQB, 128), lambda b, t: (b, t, 0)),
            pl.BlockSpec((1, N2, 3), lambda b, t: (b, 0, 0)),
            pl.BlockSpec((1, N2, 128), lambda b, t: (b, 0, 0)),
            full((128, 256)), full((1, 256)), full((128, 128)),
        ],
        out_specs=[
            pl.BlockSpec((1, TQB, 128), lambda b, t: (b, t, 0)),
            pl.BlockSpec((1, 1), lambda b, t: (0, 0)),
        ],
        out_shape=[
            jax.ShapeDtypeStruct((B, N2, 128), jnp.float32),
            jax.ShapeDtypeStruct((1, 1), jnp.float32),
        ],
    )(xyz1, f_up, enc_xyz, enc_feature, Wf0, bf0r, Wa1f)

    loss = (loss_sum[0, 0] / (B * N2 * 128)).reshape(())

    def stage_c(x1t_ref, x1f_ref, qt_ref, Wn_ref, bn_ref, Wan_ref, ba_ref,
                Wc_ref, bc_ref, c6_ref, fo_ref):
        qxyz = x1t_ref[0]
        rxyz = x1f_ref[0]
        f1 = _lfa_tile(qxyz, rxyz, qt_ref[0], Wn_ref[...], bn_ref[...],
                       Wan_ref[...], ba_ref[...], 8, 128)
        fo_ref[0] = f1
        offa = _mm(f1[:, :64], Wc_ref[...]) + bc_ref[...]
        offb = _mm(f1[:, 64:], Wc_ref[...]) + bc_ref[...]
        c6_ref[0] = jnp.concatenate([qxyz + offa, qxyz + offb], axis=1)

    c6b, f1 = pl.pallas_call(
        stage_c,
        grid=(B, nc),
        in_specs=[
            pl.BlockSpec((1, TQC, 3), lambda b, t: (b, t, 0)),
            pl.BlockSpec((1, N2, 3), lambda b, t: (b, 0, 0)),
            pl.BlockSpec((1, N2, 128), lambda b, t: (b, 0, 0)),
            full((10, 32)), full((1, 32)), full((32, 128)), full((1, 128)),
            full((64, 3)), full((1, 3)),
        ],
        out_specs=[
            pl.BlockSpec((1, TQC, 6), lambda b, t: (b, t, 0)),
            pl.BlockSpec((1, TQC, 128), lambda b, t: (b, t, 0)),
        ],
        out_shape=[
            jax.ShapeDtypeStruct((B, N2, 6), jnp.float32),
            jax.ShapeDtypeStruct((B, N2, 128), jnp.float32),
        ],
    )(xyz1, xyz1, qt, Wn1, bn1r, Wa1n, ba1r, Wc1, bc1r)

    coord2 = c6b.reshape(B, 4 * N, 3)
    fout = f1.reshape(B, 4 * N, 64)
    return (coord2, fout, loss)


# speedup vs baseline: 1.0977x; 1.0405x over previous
import functools

import jax
import jax.numpy as jnp
from jax import lax
from jax.experimental import pallas as pl
from jax.experimental.pallas import tpu as pltpu, tpu_sc as plsc


def _leaky(x):
    return jnp.where(x > 0, x, 0.2 * x)


def _rowdot(a, b):
    return jax.lax.dot_general(a, b, (((1,), (1,)), ((), ())),
                               preferred_element_type=jnp.float32)


def _mm(a, b):
    return jnp.dot(a, b, preferred_element_type=jnp.float32)


def _split3(x):
    h1 = x.astype(jnp.bfloat16).astype(jnp.float32)
    r = x - h1
    h2 = r.astype(jnp.bfloat16).astype(jnp.float32)
    h3 = r - h2
    return h1, h2, h3


def _gather(onehot, table):
    c = table.shape[1] // 3
    g = jnp.dot(onehot, table, preferred_element_type=jnp.float32)
    return (g[:, :c] + g[:, c:2 * c]) + g[:, 2 * c:]


def _tree3(p0, p1, p2):
    return (p0 + p2) + p1


def _norm3(x):
    return _tree3(x[:, 0:1] * x[:, 0:1], x[:, 1:2] * x[:, 1:2],
                  x[:, 2:3] * x[:, 2:3])


def _lfa_tile(qxyz, rxyz, ftrans, Wn, bn, Wan, ba, k, cdim):
    tq = qxyz.shape[0]
    n = rxyz.shape[0]
    q2 = _norm3(qxyz)
    r2 = _norm3(rxyz)
    d = (q2 - 2.0 * _rowdot(qxyz, rxyz)) + r2.reshape(1, n)
    fp = _split3(ftrans)
    xp = _split3(rxyz)
    tab = jnp.concatenate(list(fp) + list(xp), axis=1).astype(jnp.bfloat16)
    xb = 3 * cdim
    col = jax.lax.broadcasted_iota(jnp.int32, (tq, n), 1)
    acc = jnp.full((tq, cdim), -jnp.inf, jnp.float32)
    for _ in range(k):
        m = jnp.min(d, axis=1, keepdims=True)
        cand = jnp.where(d == m, col, n)
        sel = jnp.min(cand, axis=1, keepdims=True)
        hit = col == sel
        onehot = hit.astype(jnp.bfloat16)
        d = jnp.where(hit, jnp.inf, d)
        g = jnp.dot(onehot, tab, preferred_element_type=jnp.float32)
        h = (g[:, :cdim] + g[:, cdim:2 * cdim]) + g[:, 2 * cdim:xb]
        nbr = (g[:, xb:xb + 3] + g[:, xb + 3:xb + 6]) + g[:, xb + 6:xb + 9]
        rel = nbr - qxyz
        dist = jnp.sqrt(_tree3(rel[:, 0:1] * rel[:, 0:1],
                               rel[:, 1:2] * rel[:, 1:2],
                               rel[:, 2:3] * rel[:, 2:3]) + 1e-12)
        geo = jnp.concatenate([rel, dist, qxyz, nbr], axis=1)
        nf = _leaky(_mm(geo, Wn) + bn)
        z = (h + _mm(nf, Wan)) + ba
        acc = jnp.maximum(acc, z)
    return _leaky(acc)


def kernel(xyz, feature, enc_xyz, enc_feature, Wn0, bn0, Wa0, ba0, Wc0, bc0,
           Wf0, bf0, Wn1, bn1, Wa1, ba1, Wc1, bc1):
    B, N, _ = xyz.shape
    N2 = 2 * N
    TQ0 = N // 4
    TQB = N2 // 8
    TQC = N2 // 8

    Wa0f, Wa0n = Wa0[:256], Wa0[256:]
    Wa1f, Wa1n = Wa1[:128], Wa1[128:]
    bn0r, ba0r = bn0.reshape(1, -1), ba0.reshape(1, -1)
    bn1r, ba1r = bn1.reshape(1, -1), ba1.reshape(1, -1)
    bc0r, bf0r, bc1r = bc0.reshape(1, -1), bf0.reshape(1, -1), bc1.reshape(1, -1)

    full = lambda shape: pl.BlockSpec(shape, lambda b, t, s=len(shape): (0,) * s)

    def stage_a(xyzt_ref, xyzf_ref, feat_ref, Wn_ref, bn_ref, Waf_ref,
                Wan_ref, ba_ref, Wc_ref, bc_ref, f0_ref, c6_ref):
        qxyz = xyzt_ref[0]
        rxyz = xyzf_ref[0]
        ftrans = _mm(feat_ref[0], Waf_ref[...])
        f0 = _lfa_tile(qxyz, rxyz, ftrans, Wn_ref[...], bn_ref[...],
                       Wan_ref[...], ba_ref[...], 8, 256)
        f0_ref[0] = f0
        offa = _mm(f0[:, :128], Wc_ref[...]) + bc_ref[...]
        offb = _mm(f0[:, 128:], Wc_ref[...]) + bc_ref[...]
        c6_ref[0] = jnp.concatenate([qxyz + offa, qxyz + offb], axis=1)

    na = N // TQ0
    f0, c6 = pl.pallas_call(
        stage_a,
        grid=(B, na),
        in_specs=[
            pl.BlockSpec((1, TQ0, 3), lambda b, t: (b, t, 0)),
            pl.BlockSpec((1, N, 3), lambda b, t: (b, 0, 0)),
            pl.BlockSpec((1, N, 256), lambda b, t: (b, 0, 0)),
            full((10, 32)), full((1, 32)), full((256, 256)), full((32, 256)),
            full((1, 256)), full((128, 3)), full((1, 3)),
        ],
        out_specs=[
            pl.BlockSpec((1, TQ0, 256), lambda b, t: (b, t, 0)),
            pl.BlockSpec((1, TQ0, 6), lambda b, t: (b, t, 0)),
        ],
        out_shape=[
            jax.ShapeDtypeStruct((B, N, 256), jnp.float32),
            jax.ShapeDtypeStruct((B, N, 6), jnp.float32),
        ],
    )(xyz, xyz, feature, Wn0, bn0r, Wa0f, Wa0n, ba0r, Wc0, bc0r)

    xyz1 = c6.reshape(B, N2, 3)
    f_up = f0.reshape(B, N2, 128)

    def stage_b1(x1_ref, ex_ref, idx_ref):
        qxyz = x1_ref[0]
        e = ex_ref[0]
        tq = qxyz.shape[0]
        ne = e.shape[0]
        q2 = _norm3(qxyz)
        e2 = _norm3(e)
        d = (q2 - 2.0 * _rowdot(qxyz, e)) + e2.reshape(1, ne)
        col = jax.lax.broadcasted_iota(jnp.int32, (tq, ne), 1)
        sels = []
        for _ in range(4):
            m = jnp.min(d, axis=1, keepdims=True)
            cand = jnp.where(d == m, col, ne)
            sel = jnp.min(cand, axis=1, keepdims=True)
            sels.append(sel)
            d = jnp.where(col == sel, jnp.inf, d)
        base = pl.program_id(0) * ne
        idx_ref[0] = jnp.concatenate(sels, axis=1) + base

    nb = N2 // TQB
    nc = N2 // TQC
    idx4 = pl.pallas_call(
        stage_b1,
        grid=(B, nb),
        in_specs=[
            pl.BlockSpec((1, TQB, 3), lambda b, t: (b, t, 0)),
            pl.BlockSpec((1, N2, 3), lambda b, t: (b, 0, 0)),
        ],
        out_specs=pl.BlockSpec((1, TQB, 4), lambda b, t: (b, t, 0)),
        out_shape=jax.ShapeDtypeStruct((B, N2, 4), jnp.int32),
    )(xyz1, enc_xyz)

    NG = B * N2 * 4
    NC_, NS_ = 2, 16
    NW = NC_ * NS_
    b_per_w = NG // NW
    CH = 128
    mesh = plsc.VectorSubcoreMesh(core_axis_name="c", subcore_axis_name="s")

    @functools.partial(
        pl.kernel, mesh=mesh,
        out_type=jax.ShapeDtypeStruct((NG, 128), jnp.float32),
        scratch_types=[
            pltpu.VMEM((CH,), jnp.int32),
            pltpu.VMEM((CH, 128), jnp.float32),
            pltpu.SemaphoreType.DMA,
        ],
    )
    def sc_gather(table_hbm, idx_hbm, out_hbm, idx_v, rows_v, sem):
        wid = lax.axis_index("s") * NC_ + lax.axis_index("c")
        base = wid * b_per_w
        for i in range(b_per_w // CH):
            off = base + i * CH
            pltpu.sync_copy(idx_hbm.at[pl.ds(off, CH)], idx_v)
            pltpu.async_copy(table_hbm.at[idx_v], rows_v, sem).wait()
            pltpu.sync_copy(rows_v, out_hbm.at[pl.ds(off, CH)])

    rows4 = sc_gather(enc_feature.reshape(B * N2, 128),
                      idx4.reshape(NG)).reshape(B, N2, 512)

    def stage_b2(r4_ref, f_ref, Wf_ref, bf_ref, Waf_ref, qt_ref, loss_ref):
        r4 = r4_ref[0]
        res = jnp.maximum(jnp.maximum(r4[:, :128], r4[:, 128:256]),
                          jnp.maximum(r4[:, 256:384], r4[:, 384:512]))
        pred = _mm(f_ref[0], Wf_ref[...]) + bf_ref[...]
        loc = pred[:, :128]
        log_b = pred[:, 128:]
        bb = jnp.exp(jnp.clip(log_b, -8.0, 8.0)) + 1e-6
        qv = jnp.round(res)

        def cdf(x):
            return 0.5 + 0.5 * jnp.sign(x - loc) * (
                1.0 - jnp.exp(-jnp.abs(x - loc) / bb))

        p = jnp.clip(cdf(qv + 0.5) - cdf(qv - 0.5), 1e-9, 1.0)
        tile_loss = jnp.sum(-jnp.log2(p), keepdims=True)

        @pl.when(jnp.logical_and(pl.program_id(0) == 0, pl.program_id(1) == 0))
        def _():
            loss_ref[...] = jnp.zeros((1, 1), jnp.float32)

        loss_ref[...] += tile_loss
        qt_ref[0] = _mm(qv, Waf_ref[...])

    qt, loss_sum = pl.pallas_call(
        stage_b2,
        grid=(B, nb),
        in_specs=[
            pl.BlockSpec((1, TQB, 512), lambda b, t: (b, t, 0)),
            pl.BlockSpec((1, TQB, 128), lambda b, t: (b, t, 0)),
            full((128, 256)), full((1, 256)), full((128, 128)),
        ],
        out_specs=[
            pl.BlockSpec((1, TQB, 128), lambda b, t: (b, t, 0)),
            pl.BlockSpec((1, 1), lambda b, t: (0, 0)),
        ],
        out_shape=[
            jax.ShapeDtypeStruct((B, N2, 128), jnp.float32),
            jax.ShapeDtypeStruct((1, 1), jnp.float32),
        ],
    )(rows4, f_up, Wf0, bf0r, Wa1f)

    loss = (loss_sum[0, 0] / (B * N2 * 128)).reshape(())

    def stage_c(x1t_ref, x1f_ref, qt_ref, Wn_ref, bn_ref, Wan_ref, ba_ref,
                Wc_ref, bc_ref, c6_ref, fo_ref):
        qxyz = x1t_ref[0]
        rxyz = x1f_ref[0]
        f1 = _lfa_tile(qxyz, rxyz, qt_ref[0], Wn_ref[...], bn_ref[...],
                       Wan_ref[...], ba_ref[...], 8, 128)
        fo_ref[0] = f1
        offa = _mm(f1[:, :64], Wc_ref[...]) + bc_ref[...]
        offb = _mm(f1[:, 64:], Wc_ref[...]) + bc_ref[...]
        c6_ref[0] = jnp.concatenate([qxyz + offa, qxyz + offb], axis=1)

    c6b, f1 = pl.pallas_call(
        stage_c,
        grid=(B, nc),
        in_specs=[
            pl.BlockSpec((1, TQC, 3), lambda b, t: (b, t, 0)),
            pl.BlockSpec((1, N2, 3), lambda b, t: (b, 0, 0)),
            pl.BlockSpec((1, N2, 128), lambda b, t: (b, 0, 0)),
            full((10, 32)), full((1, 32)), full((32, 128)), full((1, 128)),
            full((64, 3)), full((1, 3)),
        ],
        out_specs=[
            pl.BlockSpec((1, TQC, 6), lambda b, t: (b, t, 0)),
            pl.BlockSpec((1, TQC, 128), lambda b, t: (b, t, 0)),
        ],
        out_shape=[
            jax.ShapeDtypeStruct((B, N2, 6), jnp.float32),
            jax.ShapeDtypeStruct((B, N2, 128), jnp.float32),
        ],
    )(xyz1, xyz1, qt, Wn1, bn1r, Wa1n, ba1r, Wc1, bc1r)

    coord2 = c6b.reshape(B, 4 * N, 3)
    fout = f1.reshape(B, 4 * N, 64)
    return (coord2, fout, loss)


# speedup vs baseline: 1.2011x; 1.0942x over previous
import functools

import jax
import jax.numpy as jnp
from jax import lax
from jax.experimental import pallas as pl
from jax.experimental.pallas import tpu as pltpu, tpu_sc as plsc


def _leaky(x):
    return jnp.where(x > 0, x, 0.2 * x)


def _rowdot(a, b):
    return jax.lax.dot_general(a, b, (((1,), (1,)), ((), ())),
                               preferred_element_type=jnp.float32)


def _mm(a, b):
    return jnp.dot(a, b, preferred_element_type=jnp.float32)


def _split3(x):
    h1 = x.astype(jnp.bfloat16).astype(jnp.float32)
    r = x - h1
    h2 = r.astype(jnp.bfloat16).astype(jnp.float32)
    h3 = r - h2
    return h1, h2, h3


def _gather(onehot, table):
    c = table.shape[1] // 3
    g = jnp.dot(onehot, table, preferred_element_type=jnp.float32)
    return (g[:, :c] + g[:, c:2 * c]) + g[:, 2 * c:]


def _tree3(p0, p1, p2):
    return (p0 + p2) + p1


def _norm3(x):
    return _tree3(x[:, 0:1] * x[:, 0:1], x[:, 1:2] * x[:, 1:2],
                  x[:, 2:3] * x[:, 2:3])


def _lfa_tile(qxyz, rxyz, ftrans, Wn, bn, Wan, ba, k, cdim):
    tq = qxyz.shape[0]
    n = rxyz.shape[0]
    q2 = _norm3(qxyz)
    r2 = _norm3(rxyz)
    d = (q2 - 2.0 * _rowdot(qxyz, rxyz)) + r2.reshape(1, n)
    fp = _split3(ftrans)
    xp = _split3(rxyz)
    tab = jnp.concatenate(list(fp) + list(xp), axis=1).astype(jnp.bfloat16)
    xb = 3 * cdim
    col = jax.lax.broadcasted_iota(jnp.int32, (tq, n), 1)
    acc = jnp.full((tq, cdim), -jnp.inf, jnp.float32)
    for _ in range(k):
        m = jnp.min(d, axis=1, keepdims=True)
        cand = jnp.where(d == m, col, n)
        sel = jnp.min(cand, axis=1, keepdims=True)
        hit = col == sel
        onehot = hit.astype(jnp.bfloat16)
        d = jnp.where(hit, jnp.inf, d)
        g = jnp.dot(onehot, tab, preferred_element_type=jnp.float32)
        h = (g[:, :cdim] + g[:, cdim:2 * cdim]) + g[:, 2 * cdim:xb]
        nbr = (g[:, xb:xb + 3] + g[:, xb + 3:xb + 6]) + g[:, xb + 6:xb + 9]
        rel = nbr - qxyz
        dist = jnp.sqrt(_tree3(rel[:, 0:1] * rel[:, 0:1],
                               rel[:, 1:2] * rel[:, 1:2],
                               rel[:, 2:3] * rel[:, 2:3]) + 1e-12)
        geo = jnp.concatenate([rel, dist, qxyz, nbr], axis=1)
        nf = _leaky(_mm(geo, Wn) + bn)
        z = (h + _mm(nf, Wan)) + ba
        acc = jnp.maximum(acc, z)
    return _leaky(acc)


def kernel(xyz, feature, enc_xyz, enc_feature, Wn0, bn0, Wa0, ba0, Wc0, bc0,
           Wf0, bf0, Wn1, bn1, Wa1, ba1, Wc1, bc1):
    B, N, _ = xyz.shape
    N2 = 2 * N
    TQ0 = N // 4
    TQB = N2 // 8
    TQC = N2 // 8

    Wa0f, Wa0n = Wa0[:256], Wa0[256:]
    Wa1f, Wa1n = Wa1[:128], Wa1[128:]
    bn0r, ba0r = bn0.reshape(1, -1), ba0.reshape(1, -1)
    bn1r, ba1r = bn1.reshape(1, -1), ba1.reshape(1, -1)
    bc0r, bf0r, bc1r = bc0.reshape(1, -1), bf0.reshape(1, -1), bc1.reshape(1, -1)

    full = lambda shape: pl.BlockSpec(shape, lambda b, t, s=len(shape): (0,) * s)

    def stage_a(xyzt_ref, xyzf_ref, feat_ref, Wn_ref, bn_ref, Waf_ref,
                Wan_ref, ba_ref, Wc_ref, bc_ref, f0_ref, c6_ref):
        qxyz = xyzt_ref[0]
        rxyz = xyzf_ref[0]
        ftrans = _mm(feat_ref[0], Waf_ref[...])
        f0 = _lfa_tile(qxyz, rxyz, ftrans, Wn_ref[...], bn_ref[...],
                       Wan_ref[...], ba_ref[...], 8, 256)
        f0_ref[0] = f0
        offa = _mm(f0[:, :128], Wc_ref[...]) + bc_ref[...]
        offb = _mm(f0[:, 128:], Wc_ref[...]) + bc_ref[...]
        c6_ref[0] = jnp.concatenate([qxyz + offa, qxyz + offb], axis=1)

    na = N // TQ0
    f0, c6 = pl.pallas_call(
        stage_a,
        grid=(B, na),
        in_specs=[
            pl.BlockSpec((1, TQ0, 3), lambda b, t: (b, t, 0)),
            pl.BlockSpec((1, N, 3), lambda b, t: (b, 0, 0)),
            pl.BlockSpec((1, N, 256), lambda b, t: (b, 0, 0)),
            full((10, 32)), full((1, 32)), full((256, 256)), full((32, 256)),
            full((1, 256)), full((128, 3)), full((1, 3)),
        ],
        out_specs=[
            pl.BlockSpec((1, TQ0, 256), lambda b, t: (b, t, 0)),
            pl.BlockSpec((1, TQ0, 6), lambda b, t: (b, t, 0)),
        ],
        out_shape=[
            jax.ShapeDtypeStruct((B, N, 256), jnp.float32),
            jax.ShapeDtypeStruct((B, N, 6), jnp.float32),
        ],
    )(xyz, xyz, feature, Wn0, bn0r, Wa0f, Wa0n, ba0r, Wc0, bc0r)

    xyz1 = c6.reshape(B, N2, 3)
    f_up = f0.reshape(B, N2, 128)

    def stage_b1(x1_ref, ex_ref, idx_ref):
        qxyz = x1_ref[0]
        e = ex_ref[0]
        tq = qxyz.shape[0]
        ne = e.shape[0]
        q2 = _norm3(qxyz)
        e2 = _norm3(e)
        d = (q2 - 2.0 * _rowdot(qxyz, e)) + e2.reshape(1, ne)
        col = jax.lax.broadcasted_iota(jnp.int32, (tq, ne), 1)
        sels = []
        for _ in range(4):
            m = jnp.min(d, axis=1, keepdims=True)
            cand = jnp.where(d == m, col, ne)
            sel = jnp.min(cand, axis=1, keepdims=True)
            sels.append(sel)
            d = jnp.where(col == sel, jnp.inf, d)
        base = pl.program_id(0) * ne
        idx_ref[0] = jnp.concatenate(sels, axis=1) + base

    nb = N2 // TQB
    nc = N2 // TQC
    idx4 = pl.pallas_call(
        stage_b1,
        grid=(B, nb),
        in_specs=[
            pl.BlockSpec((1, TQB, 3), lambda b, t: (b, t, 0)),
            pl.BlockSpec((1, N2, 3), lambda b, t: (b, 0, 0)),
        ],
        out_specs=pl.BlockSpec((1, TQB, 4), lambda b, t: (b, t, 0)),
        out_shape=jax.ShapeDtypeStruct((B, N2, 4), jnp.int32),
    )(xyz1, enc_xyz)

    NG = B * N2 * 4
    NC_, NS_ = 2, 16
    NW = NC_ * NS_
    b_per_w = NG // NW
    CH = 128
    mesh = plsc.VectorSubcoreMesh(core_axis_name="c", subcore_axis_name="s")

    @functools.partial(
        pl.kernel, mesh=mesh,
        out_type=jax.ShapeDtypeStruct((NG, 128), jnp.float32),
        scratch_types=[
            pltpu.VMEM((CH,), jnp.int32),
            pltpu.VMEM((CH, 128), jnp.float32),
            pltpu.SemaphoreType.DMA,
        ],
    )
    def sc_gather(table_hbm, idx_hbm, out_hbm, idx_v, rows_v, sem):
        wid = lax.axis_index("s") * NC_ + lax.axis_index("c")
        base = wid * b_per_w
        for i in range(b_per_w // CH):
            off = base + i * CH
            pltpu.sync_copy(idx_hbm.at[pl.ds(off, CH)], idx_v)
            pltpu.async_copy(table_hbm.at[idx_v], rows_v, sem).wait()
            pltpu.sync_copy(rows_v, out_hbm.at[pl.ds(off, CH)])

    rows4 = sc_gather(enc_feature.reshape(B * N2, 128),
                      idx4.reshape(NG)).reshape(B, N2, 512)

    def stage_b2(r4_ref, f_ref, Wf_ref, bf_ref, Waf_ref, qt_ref, loss_ref):
        r4 = r4_ref[0]
        res = jnp.maximum(jnp.maximum(r4[:, :128], r4[:, 128:256]),
                          jnp.maximum(r4[:, 256:384], r4[:, 384:512]))
        pred = _mm(f_ref[0], Wf_ref[...]) + bf_ref[...]
        loc = pred[:, :128]
        log_b = pred[:, 128:]
        bb = jnp.exp(jnp.clip(log_b, -8.0, 8.0)) + 1e-6
        qv = jnp.round(res)

        def cdf(x):
            return 0.5 + 0.5 * jnp.sign(x - loc) * (
                1.0 - jnp.exp(-jnp.abs(x - loc) / bb))

        p = jnp.clip(cdf(qv + 0.5) - cdf(qv - 0.5), 1e-9, 1.0)
        tile_loss = jnp.sum(-jnp.log2(p), keepdims=True)

        @pl.when(jnp.logical_and(pl.program_id(0) == 0, pl.program_id(1) == 0))
        def _():
            loss_ref[...] = jnp.zeros((1, 1), jnp.float32)

        loss_ref[...] += tile_loss
        qt_ref[0] = _mm(qv, Waf_ref[...])

    qt, loss_sum = pl.pallas_call(
        stage_b2,
        grid=(B, nb),
        in_specs=[
            pl.BlockSpec((1, TQB, 512), lambda b, t: (b, t, 0)),
            pl.BlockSpec((1, TQB, 128), lambda b, t: (b, t, 0)),
            full((128, 256)), full((1, 256)), full((128, 128)),
        ],
        out_specs=[
            pl.BlockSpec((1, TQB, 128), lambda b, t: (b, t, 0)),
            pl.BlockSpec((1, 1), lambda b, t: (0, 0)),
        ],
        out_shape=[
            jax.ShapeDtypeStruct((B, N2, 128), jnp.float32),
            jax.ShapeDtypeStruct((1, 1), jnp.float32),
        ],
    )(rows4, f_up, Wf0, bf0r, Wa1f)

    loss = (loss_sum[0, 0] / (B * N2 * 128)).reshape(())

    def stage_c1(x1t_ref, x1f_ref, idx_ref, nbr_ref):
        qxyz = x1t_ref[0]
        rxyz = x1f_ref[0]
        tq = qxyz.shape[0]
        n = rxyz.shape[0]
        q2 = _norm3(qxyz)
        r2 = _norm3(rxyz)
        d = (q2 - 2.0 * _rowdot(qxyz, rxyz)) + r2.reshape(1, n)
        col = jax.lax.broadcasted_iota(jnp.int32, (tq, n), 1)
        xtab = jnp.concatenate(list(_split3(rxyz)), axis=1).astype(jnp.bfloat16)
        sels = []
        nbrs = []
        for _ in range(8):
            m = jnp.min(d, axis=1, keepdims=True)
            cand = jnp.where(d == m, col, n)
            sel = jnp.min(cand, axis=1, keepdims=True)
            sels.append(sel)
            hit = col == sel
            d = jnp.where(hit, jnp.inf, d)
            g = jnp.dot(hit.astype(jnp.bfloat16), xtab,
                        preferred_element_type=jnp.float32)
            nbrs.append((g[:, 0:3] + g[:, 3:6]) + g[:, 6:9])
        base = pl.program_id(0) * n
        idx_ref[0] = jnp.concatenate(sels, axis=1) + base
        nbr_ref[0] = jnp.concatenate(nbrs, axis=1)

    idx8, nbr24 = pl.pallas_call(
        stage_c1,
        grid=(B, nc),
        in_specs=[
            pl.BlockSpec((1, TQC, 3), lambda b, t: (b, t, 0)),
            pl.BlockSpec((1, N2, 3), lambda b, t: (b, 0, 0)),
        ],
        out_specs=[
            pl.BlockSpec((1, TQC, 8), lambda b, t: (b, t, 0)),
            pl.BlockSpec((1, TQC, 24), lambda b, t: (b, t, 0)),
        ],
        out_shape=[
            jax.ShapeDtypeStruct((B, N2, 8), jnp.int32),
            jax.ShapeDtypeStruct((B, N2, 24), jnp.float32),
        ],
    )(xyz1, xyz1)

    DC = 128
    NG2 = B * N2 * 8
    bw2 = NG2 // NW

    @functools.partial(
        pl.kernel, mesh=mesh,
        out_type=jax.ShapeDtypeStruct((NG2, DC), jnp.float32),
        scratch_types=[
            pltpu.VMEM((CH,), jnp.int32),
            pltpu.VMEM((CH, DC), jnp.float32),
            pltpu.SemaphoreType.DMA,
        ],
    )
    def sc_gather8(table_hbm, idx_hbm, out_hbm, idx_v, rows_v, sem):
        wid = lax.axis_index("s") * NC_ + lax.axis_index("c")
        base = wid * bw2
        for i in range(bw2 // CH):
            off = base + i * CH
            pltpu.sync_copy(idx_hbm.at[pl.ds(off, CH)], idx_v)
            pltpu.async_copy(table_hbm.at[idx_v], rows_v, sem).wait()
            pltpu.sync_copy(rows_v, out_hbm.at[pl.ds(off, CH)])

    rows8 = sc_gather8(qt.reshape(B * N2, DC),
                       idx8.reshape(NG2)).reshape(B, N2, 8 * DC)

    def stage_c2(x1t_ref, r8_ref, nbr_ref, Wn_ref, bn_ref, Wan_ref, ba_ref,
                 Wc_ref, bc_ref, c6_ref, fo_ref):
        qxyz = x1t_ref[0]
        r8 = r8_ref[0]
        nbr24 = nbr_ref[0]
        acc = jnp.full((qxyz.shape[0], 128), -jnp.inf, jnp.float32)
        for j in range(8):
            o = j * DC
            h = r8[:, o:o + 128]
            nbr = nbr24[:, 3 * j:3 * j + 3]
            rel = nbr - qxyz
            dist = jnp.sqrt(_tree3(rel[:, 0:1] * rel[:, 0:1],
                                   rel[:, 1:2] * rel[:, 1:2],
                                   rel[:, 2:3] * rel[:, 2:3]) + 1e-12)
            geo = jnp.concatenate([rel, dist, qxyz, nbr], axis=1)
            nf = _leaky(_mm(geo, Wn_ref[...]) + bn_ref[...])
            z = (h + _mm(nf, Wan_ref[...])) + ba_ref[...]
            acc = jnp.maximum(acc, z)
        f1 = _leaky(acc)
        fo_ref[0] = f1
        offa = _mm(f1[:, :64], Wc_ref[...]) + bc_ref[...]
        offb = _mm(f1[:, 64:], Wc_ref[...]) + bc_ref[...]
        c6_ref[0] = jnp.concatenate([qxyz + offa, qxyz + offb], axis=1)

    c6b, f1 = pl.pallas_call(
        stage_c2,
        grid=(B, nc),
        in_specs=[
            pl.BlockSpec((1, TQC, 3), lambda b, t: (b, t, 0)),
            pl.BlockSpec((1, TQC, 8 * DC), lambda b, t: (b, t, 0)),
            pl.BlockSpec((1, TQC, 24), lambda b, t: (b, t, 0)),
            full((10, 32)), full((1, 32)), full((32, 128)), full((1, 128)),
            full((64, 3)), full((1, 3)),
        ],
        out_specs=[
            pl.BlockSpec((1, TQC, 6), lambda b, t: (b, t, 0)),
            pl.BlockSpec((1, TQC, 128), lambda b, t: (b, t, 0)),
        ],
        out_shape=[
            jax.ShapeDtypeStruct((B, N2, 6), jnp.float32),
            jax.ShapeDtypeStruct((B, N2, 128), jnp.float32),
        ],
    )(xyz1, rows8, nbr24, Wn1, bn1r, Wa1n, ba1r, Wc1, bc1r)

    coord2 = c6b.reshape(B, 4 * N, 3)
    fout = f1.reshape(B, 4 * N, 64)
    return (coord2, fout, loss)
